# baseline (speedup 1.0000x reference)
import functools

import jax
import jax.numpy as jnp
from jax import lax
from jax.experimental import pallas as pl
from jax.experimental.pallas import tpu as pltpu
from jax.experimental.pallas import tpu_sc as plsc

N_LAST = 8


def kernel(x, seq_len):
    B, T, F = x.shape
    x2d = x.reshape(B * T, F)
    mesh = plsc.VectorSubcoreMesh(
        core_axis_name="c", subcore_axis_name="s", num_cores=1
    )

    @functools.partial(
        pl.kernel,
        out_type=jax.ShapeDtypeStruct((B * N_LAST, F), x.dtype),
        mesh=mesh,
        scratch_types=[
            pltpu.VMEM((16,), jnp.int32),
            pltpu.VMEM((16, F), jnp.float32),
            pltpu.SemaphoreType.DMA,
        ],
        compiler_params=pltpu.CompilerParams(needs_layout_passes=False),
    )
    def take_last(x_hbm, seq_hbm, out_hbm, seq_v, rows_v, sem):
        b = lax.axis_index("s")
        pltpu.sync_copy(seq_hbm, seq_v)
        lane = lax.iota(jnp.int32, 16)
        len_b = plsc.load_gather(seq_v, [jnp.full((16,), b, jnp.int32)])
        row_ids = b * T - N_LAST + len_b + jnp.minimum(lane, N_LAST - 1)
        pltpu.async_copy(x_hbm.at[row_ids], rows_v, sem).wait()
        pltpu.sync_copy(
            rows_v.at[pl.ds(0, N_LAST)],
            out_hbm.at[pl.ds(pl.multiple_of(b * N_LAST, 8), N_LAST)],
        )

    out = take_last(x2d, seq_len)
    return out.reshape(B, N_LAST, F)


# speedup vs baseline: 1.0972x; 1.0972x over previous
import functools

import jax
import jax.numpy as jnp
from jax import lax
from jax.experimental import pallas as pl
from jax.experimental.pallas import tpu as pltpu
from jax.experimental.pallas import tpu_sc as plsc

N_LAST = 8


def kernel(x, seq_len):
    B, T, F = x.shape
    x2d = x.reshape(B * T, F)
    row_ids = (
        jnp.arange(B, dtype=jnp.int32) * T + seq_len - N_LAST
    )[:, None] + jnp.arange(N_LAST, dtype=jnp.int32)[None, :]
    mesh = plsc.VectorSubcoreMesh(
        core_axis_name="c", subcore_axis_name="s", num_cores=1
    )

    @functools.partial(
        pl.kernel,
        out_type=jax.ShapeDtypeStruct((B * N_LAST, F), x.dtype),
        mesh=mesh,
        scratch_types=[
            pltpu.VMEM((N_LAST,), jnp.int32),
            pltpu.VMEM((N_LAST, F), jnp.float32),
            pltpu.SemaphoreType.DMA,
        ],
        compiler_params=pltpu.CompilerParams(needs_layout_passes=False),
    )
    def take_last(x_hbm, ids_hbm, out_hbm, idx_v, rows_v, sem):
        b = lax.axis_index("s")
        base = pl.multiple_of(b * N_LAST, 8)
        pltpu.sync_copy(ids_hbm.at[pl.ds(base, N_LAST)], idx_v)
        pltpu.async_copy(x_hbm.at[idx_v], rows_v, sem).wait()
        pltpu.sync_copy(rows_v, out_hbm.at[pl.ds(base, N_LAST)])

    out = take_last(x2d, row_ids.reshape(B * N_LAST))
    return out.reshape(B, N_LAST, F)
